# initial kernel scaffold (unmeasured)
import jax
import jax.numpy as jnp
from jax import lax
from jax.experimental import pallas as pl
from jax.experimental.pallas import tpu as pltpu

N_DEV = 8
HQ_PER = 8
DH = 128
SQ = 1024
D_MODEL = 1024
BLK = 64
N_CLASS = 4
BLOCKS_PER_CLASS = 4
CHUNK = SQ // N_DEV
SCALE = 0.08838834764831843
N_HOP = N_DEV - 1


def kernel(x, Wq, K_ext, V_ext, Wo):
    def body(x_ref, wq_ref, k_hbm, v_hbm, wo_ref, out_ref,
             k_ref, v_ref, q_ref, ctx_ref, recv_ref,
             copy_sems, send_sems, recv_sems):
        me = lax.axis_index("i")
        left = lax.rem(me + N_DEV - 1, N_DEV)
        right = lax.rem(me + 1, N_DEV)
        h0 = me * HQ_PER

        def mod(v):
            return lax.rem(v + 2 * N_DEV, N_DEV)

        copies = []
        for h in range(HQ_PER):
            kc = pltpu.make_async_copy(
                k_hbm.at[0, :, h0 + h, :], k_ref.at[h], copy_sems.at[h])
            vc = pltpu.make_async_copy(
                v_hbm.at[0, :, h0 + h, :], v_ref.at[h],
                copy_sems.at[HQ_PER + h])
            kc.start()
            vc.start()
            copies += [kc, vc]

        barrier = pltpu.get_barrier_semaphore()
        for nbr in (left, right):
            pl.semaphore_signal(barrier, inc=1, device_id=(nbr,),
                                device_id_type=pl.DeviceIdType.MESH)
        pl.semaphore_wait(barrier, 2)

        q_ref[:, :] = jnp.dot(x_ref[0], wq_ref[:, :],
                              preferred_element_type=jnp.float32)

        for c in copies:
            c.wait()

        for c in range(N_CLASS):
            rows = [(c + N_CLASS * b) * BLK for b in range(BLOCKS_PER_CLASS)]
            for h in range(HQ_PER):
                qc = jnp.concatenate(
                    [q_ref[r:r + BLK, h * DH:(h + 1) * DH] for r in rows], 0)
                kc = jnp.concatenate(
                    [k_ref[h, r:r + BLK, :] for r in rows], 0)
                vc = jnp.concatenate(
                    [v_ref[h, r:r + BLK, :] for r in rows], 0)
                s = lax.dot_general(
                    qc, kc, (((1,), (1,)), ((), ())),
                    preferred_element_type=jnp.float32) * SCALE
                m = jnp.max(s, axis=1, keepdims=True)
                e = jnp.exp(s - m)
                w = e / jnp.sum(e, axis=1, keepdims=True)
                ctx = jnp.dot(w, vc, preferred_element_type=jnp.float32)
                for b, r in enumerate(rows):
                    ctx_ref[r:r + BLK, h * DH:(h + 1) * DH] = \
                        ctx[b * BLK:(b + 1) * BLK, :]

        o_ref = out_ref.at[0]
        o_ref[:, :] = jnp.dot(ctx_ref[:, :], wo_ref[:, :],
                              preferred_element_type=jnp.float32)

        for s in range(N_HOP):
            c_send = mod(me - s)
            c_recv = mod(me - s - 1)
            rdma = pltpu.make_async_remote_copy(
                src_ref=o_ref.at[pl.ds(c_send * CHUNK, CHUNK), :],
                dst_ref=recv_ref.at[s],
                send_sem=send_sems.at[s],
                recv_sem=recv_sems.at[s],
                device_id=(right,),
                device_id_type=pl.DeviceIdType.MESH,
            )
            rdma.start()
            rdma.wait()
            r0 = c_recv * CHUNK
            o_ref[pl.ds(r0, CHUNK), :] = (
                o_ref[pl.ds(r0, CHUNK), :] + recv_ref[s])

        for s in range(N_HOP):
            c_send = mod(me + 1 - s)
            rdma = pltpu.make_async_remote_copy(
                src_ref=o_ref.at[pl.ds(c_send * CHUNK, CHUNK), :],
                dst_ref=o_ref.at[pl.ds(c_send * CHUNK, CHUNK), :],
                send_sem=send_sems.at[N_HOP + s],
                recv_sem=recv_sems.at[N_HOP + s],
                device_id=(right,),
                device_id_type=pl.DeviceIdType.MESH,
            )
            rdma.start()
            rdma.wait()

    return pl.pallas_call(
        body,
        out_shape=jax.ShapeDtypeStruct((1, SQ, D_MODEL), jnp.float32),
        in_specs=[
            pl.BlockSpec(memory_space=pltpu.VMEM),
            pl.BlockSpec(memory_space=pltpu.VMEM),
            pl.BlockSpec(memory_space=pltpu.ANY),
            pl.BlockSpec(memory_space=pltpu.ANY),
            pl.BlockSpec(memory_space=pltpu.VMEM),
        ],
        out_specs=pl.BlockSpec(memory_space=pltpu.VMEM),
        scratch_shapes=[
            pltpu.VMEM((HQ_PER, SQ, DH), jnp.float32),
            pltpu.VMEM((HQ_PER, SQ, DH), jnp.float32),
            pltpu.VMEM((SQ, HQ_PER * DH), jnp.float32),
            pltpu.VMEM((SQ, HQ_PER * DH), jnp.float32),
            pltpu.VMEM((N_HOP, CHUNK, D_MODEL), jnp.float32),
            pltpu.SemaphoreType.DMA((2 * HQ_PER,)),
            pltpu.SemaphoreType.DMA((2 * N_HOP,)),
            pltpu.SemaphoreType.DMA((2 * N_HOP,)),
        ],
        compiler_params=pltpu.CompilerParams(collective_id=0),
    )(x, Wq, K_ext, V_ext, Wo)


# baseline (device time: 124555 ns/iter reference)
import jax
import jax.numpy as jnp
from jax import lax
from jax.experimental import pallas as pl
from jax.experimental.pallas import tpu as pltpu

N_DEV = 8
HQ_PER = 8
DH = 128
SQ = 1024
D_MODEL = 1024
BLK = 64
N_CLASS = 4
BLOCKS_PER_CLASS = 4
CHUNK = SQ // N_DEV
SCALE = 0.08838834764831843
N_HOP = N_DEV - 1


def kernel(x, Wq, K_ext, V_ext, Wo):
    def body(x_ref, wq_ref, k_hbm, v_hbm, wo_ref, out_ref,
             k_ref, v_ref, q_ref, ctx_ref, recv_ref,
             copy_sems, send_sems, recv_sems):
        me = lax.axis_index("i")
        left = lax.rem(me + N_DEV - 1, N_DEV)
        right = lax.rem(me + 1, N_DEV)
        h0 = me * HQ_PER

        def mod(v):
            return lax.rem(v + 2 * N_DEV, N_DEV)

        copies = []
        for h in range(HQ_PER):
            kc = pltpu.make_async_copy(
                k_hbm.at[0, :, h0 + h, :], k_ref.at[h], copy_sems.at[h])
            vc = pltpu.make_async_copy(
                v_hbm.at[0, :, h0 + h, :], v_ref.at[h],
                copy_sems.at[HQ_PER + h])
            kc.start()
            vc.start()
            copies += [kc, vc]

        barrier = pltpu.get_barrier_semaphore()
        for nbr in (left, right):
            pl.semaphore_signal(barrier, inc=1, device_id=(nbr,),
                                device_id_type=pl.DeviceIdType.MESH)
        pl.semaphore_wait(barrier, 2)

        q_ref[:, :] = jnp.dot(x_ref[0], wq_ref[:, :],
                              preferred_element_type=jnp.float32)

        for c in copies:
            c.wait()

        for c in range(N_CLASS):
            rows = [(c + N_CLASS * b) * BLK for b in range(BLOCKS_PER_CLASS)]
            for h in range(HQ_PER):
                qc = jnp.concatenate(
                    [q_ref[r:r + BLK, h * DH:(h + 1) * DH] for r in rows], 0)
                kc = jnp.concatenate(
                    [k_ref[h, r:r + BLK, :] for r in rows], 0)
                vc = jnp.concatenate(
                    [v_ref[h, r:r + BLK, :] for r in rows], 0)
                s = lax.dot_general(
                    qc, kc, (((1,), (1,)), ((), ())),
                    preferred_element_type=jnp.float32) * SCALE
                m = jnp.max(s, axis=1, keepdims=True)
                e = jnp.exp(s - m)
                w = e / jnp.sum(e, axis=1, keepdims=True)
                ctx = jnp.dot(w, vc, preferred_element_type=jnp.float32)
                for b, r in enumerate(rows):
                    ctx_ref[r:r + BLK, h * DH:(h + 1) * DH] = \
                        ctx[b * BLK:(b + 1) * BLK, :]

        o_ref = out_ref.at[0]
        o_ref[:, :] = jnp.dot(ctx_ref[:, :], wo_ref[:, :],
                              preferred_element_type=jnp.float32)

        for s in range(N_HOP):
            c_send = mod(me - s)
            c_recv = mod(me - s - 1)
            rdma = pltpu.make_async_remote_copy(
                src_ref=o_ref.at[pl.ds(c_send * CHUNK, CHUNK), :],
                dst_ref=recv_ref.at[s],
                send_sem=send_sems.at[s],
                recv_sem=recv_sems.at[s],
                device_id=(right,),
                device_id_type=pl.DeviceIdType.MESH,
            )
            rdma.start()
            rdma.wait()
            r0 = c_recv * CHUNK
            o_ref[pl.ds(r0, CHUNK), :] = (
                o_ref[pl.ds(r0, CHUNK), :] + recv_ref[s])

        for s in range(N_HOP):
            c_send = mod(me + 1 - s)
            rdma = pltpu.make_async_remote_copy(
                src_ref=o_ref.at[pl.ds(c_send * CHUNK, CHUNK), :],
                dst_ref=o_ref.at[pl.ds(c_send * CHUNK, CHUNK), :],
                send_sem=send_sems.at[N_HOP + s],
                recv_sem=recv_sems.at[N_HOP + s],
                device_id=(right,),
                device_id_type=pl.DeviceIdType.MESH,
            )
            rdma.start()
            rdma.wait()

    return pl.pallas_call(
        body,
        out_shape=jax.ShapeDtypeStruct((1, SQ, D_MODEL), jnp.float32),
        in_specs=[
            pl.BlockSpec(memory_space=pltpu.VMEM),
            pl.BlockSpec(memory_space=pltpu.VMEM),
            pl.BlockSpec(memory_space=pl.ANY),
            pl.BlockSpec(memory_space=pl.ANY),
            pl.BlockSpec(memory_space=pltpu.VMEM),
        ],
        out_specs=pl.BlockSpec(memory_space=pltpu.VMEM),
        scratch_shapes=[
            pltpu.VMEM((HQ_PER, SQ, DH), jnp.float32),
            pltpu.VMEM((HQ_PER, SQ, DH), jnp.float32),
            pltpu.VMEM((SQ, HQ_PER * DH), jnp.float32),
            pltpu.VMEM((SQ, HQ_PER * DH), jnp.float32),
            pltpu.VMEM((N_HOP, CHUNK, D_MODEL), jnp.float32),
            pltpu.SemaphoreType.DMA((2 * HQ_PER,)),
            pltpu.SemaphoreType.DMA((2 * N_HOP,)),
            pltpu.SemaphoreType.DMA((2 * N_HOP,)),
        ],
        compiler_params=pltpu.CompilerParams(collective_id=0),
    )(x, Wq, K_ext, V_ext, Wo)


# device time: 88884 ns/iter; 1.4013x vs baseline; 1.4013x over previous
import jax
import jax.numpy as jnp
from jax import lax
from jax.experimental import pallas as pl
from jax.experimental.pallas import tpu as pltpu

N_DEV = 8
HQ_PER = 8
DH = 128
SQ = 1024
D_MODEL = 1024
BLK = 64
N_CLASS = 4
BLOCKS_PER_CLASS = 4
CHUNK = SQ // N_DEV
HALF = D_MODEL // 2
SCALE = 0.08838834764831843
N_HOP = N_DEV - 1


def kernel(x, Wq, K_ext, V_ext, Wo):
    def body(x_ref, wq_ref, k_hbm, v_hbm, wo_ref, out_ref,
             k_ref, v_ref, q_ref, ctx_ref, recv_r_ref, recv_l_ref,
             copy_sems, send_r_sems, recv_r_sems, send_l_sems, recv_l_sems):
        me = lax.axis_index("i")
        left = lax.rem(me + N_DEV - 1, N_DEV)
        right = lax.rem(me + 1, N_DEV)
        h0 = me * HQ_PER

        def mod(v):
            return lax.rem(v + 2 * N_DEV, N_DEV)

        copies = []
        for h in range(HQ_PER):
            kc = pltpu.make_async_copy(
                k_hbm.at[0, :, h0 + h, :], k_ref.at[h], copy_sems.at[h])
            vc = pltpu.make_async_copy(
                v_hbm.at[0, :, h0 + h, :], v_ref.at[h],
                copy_sems.at[HQ_PER + h])
            kc.start()
            vc.start()
            copies += [kc, vc]

        barrier = pltpu.get_barrier_semaphore()
        for nbr in (left, right):
            pl.semaphore_signal(barrier, inc=1, device_id=(nbr,),
                                device_id_type=pl.DeviceIdType.MESH)
        pl.semaphore_wait(barrier, 2)

        q_ref[:, :] = jnp.dot(x_ref[0], wq_ref[:, :],
                              preferred_element_type=jnp.float32)

        for c in copies:
            c.wait()

        for c in range(N_CLASS):
            rows = [(c + N_CLASS * b) * BLK for b in range(BLOCKS_PER_CLASS)]
            for h in range(HQ_PER):
                qc = jnp.concatenate(
                    [q_ref[r:r + BLK, h * DH:(h + 1) * DH] for r in rows], 0)
                kc = jnp.concatenate(
                    [k_ref[h, r:r + BLK, :] for r in rows], 0)
                vc = jnp.concatenate(
                    [v_ref[h, r:r + BLK, :] for r in rows], 0)
                s = lax.dot_general(
                    qc, kc, (((1,), (1,)), ((), ())),
                    preferred_element_type=jnp.float32) * SCALE
                m = jnp.max(s, axis=1, keepdims=True)
                e = jnp.exp(s - m)
                w = e / jnp.sum(e, axis=1, keepdims=True)
                ctx = jnp.dot(w, vc, preferred_element_type=jnp.float32)
                for b, r in enumerate(rows):
                    ctx_ref[r:r + BLK, h * DH:(h + 1) * DH] = \
                        ctx[b * BLK:(b + 1) * BLK, :]

        o_ref = out_ref.at[0]

        def gemm(chunk, col0):
            r0 = chunk * CHUNK
            o_ref[pl.ds(r0, CHUNK), col0:col0 + HALF] = jnp.dot(
                ctx_ref[pl.ds(r0, CHUNK), :], wo_ref[:, col0:col0 + HALF],
                preferred_element_type=jnp.float32)

        def ring_copy(chunk, col0, dst, nbr, ssem, rsem):
            r0 = chunk * CHUNK
            return pltpu.make_async_remote_copy(
                src_ref=o_ref.at[pl.ds(r0, CHUNK), col0:col0 + HALF],
                dst_ref=dst,
                send_sem=ssem,
                recv_sem=rsem,
                device_id=(nbr,),
                device_id_type=pl.DeviceIdType.MESH,
            )

        gemm(me, 0)
        gemm(me, HALF)
        hop_r = ring_copy(me, 0, recv_r_ref.at[0], right,
                          send_r_sems.at[0], recv_r_sems.at[0])
        hop_l = ring_copy(me, HALF, recv_l_ref.at[0], left,
                          send_l_sems.at[0], recv_l_sems.at[0])
        hop_r.start()
        hop_l.start()
        for s in range(N_HOP):
            c_r = mod(me - s - 1)
            c_l = mod(me + s + 1)
            gemm(c_r, 0)
            gemm(c_l, HALF)
            hop_r.wait()
            r0 = c_r * CHUNK
            o_ref[pl.ds(r0, CHUNK), 0:HALF] = (
                o_ref[pl.ds(r0, CHUNK), 0:HALF] + recv_r_ref[s])
            hop_l.wait()
            r0 = c_l * CHUNK
            o_ref[pl.ds(r0, CHUNK), HALF:D_MODEL] = (
                o_ref[pl.ds(r0, CHUNK), HALF:D_MODEL] + recv_l_ref[s])
            if s < N_HOP - 1:
                hop_r = ring_copy(c_r, 0, recv_r_ref.at[s + 1], right,
                                  send_r_sems.at[s + 1], recv_r_sems.at[s + 1])
                hop_l = ring_copy(c_l, HALF, recv_l_ref.at[s + 1], left,
                                  send_l_sems.at[s + 1], recv_l_sems.at[s + 1])
                hop_r.start()
                hop_l.start()

        def ag_copy(chunk, col0, nbr, ssem, rsem):
            r0 = chunk * CHUNK
            return pltpu.make_async_remote_copy(
                src_ref=o_ref.at[pl.ds(r0, CHUNK), col0:col0 + HALF],
                dst_ref=o_ref.at[pl.ds(r0, CHUNK), col0:col0 + HALF],
                send_sem=ssem,
                recv_sem=rsem,
                device_id=(nbr,),
                device_id_type=pl.DeviceIdType.MESH,
            )

        hop_r = ag_copy(mod(me + 1), 0, right,
                        send_r_sems.at[N_HOP], recv_r_sems.at[N_HOP])
        hop_l = ag_copy(mod(me - 1), HALF, left,
                        send_l_sems.at[N_HOP], recv_l_sems.at[N_HOP])
        hop_r.start()
        hop_l.start()
        for s in range(N_HOP):
            hop_r.wait()
            hop_l.wait()
            if s < N_HOP - 1:
                hop_r = ag_copy(mod(me - s), 0, right,
                                send_r_sems.at[N_HOP + s + 1],
                                recv_r_sems.at[N_HOP + s + 1])
                hop_l = ag_copy(mod(me + s), HALF, left,
                                send_l_sems.at[N_HOP + s + 1],
                                recv_l_sems.at[N_HOP + s + 1])
                hop_r.start()
                hop_l.start()

    return pl.pallas_call(
        body,
        out_shape=jax.ShapeDtypeStruct((1, SQ, D_MODEL), jnp.float32),
        in_specs=[
            pl.BlockSpec(memory_space=pltpu.VMEM),
            pl.BlockSpec(memory_space=pltpu.VMEM),
            pl.BlockSpec(memory_space=pl.ANY),
            pl.BlockSpec(memory_space=pl.ANY),
            pl.BlockSpec(memory_space=pltpu.VMEM),
        ],
        out_specs=pl.BlockSpec(memory_space=pltpu.VMEM),
        scratch_shapes=[
            pltpu.VMEM((HQ_PER, SQ, DH), jnp.float32),
            pltpu.VMEM((HQ_PER, SQ, DH), jnp.float32),
            pltpu.VMEM((SQ, HQ_PER * DH), jnp.float32),
            pltpu.VMEM((SQ, HQ_PER * DH), jnp.float32),
            pltpu.VMEM((N_HOP, CHUNK, HALF), jnp.float32),
            pltpu.VMEM((N_HOP, CHUNK, HALF), jnp.float32),
            pltpu.SemaphoreType.DMA((2 * HQ_PER,)),
            pltpu.SemaphoreType.DMA((2 * N_HOP,)),
            pltpu.SemaphoreType.DMA((2 * N_HOP,)),
            pltpu.SemaphoreType.DMA((2 * N_HOP,)),
            pltpu.SemaphoreType.DMA((2 * N_HOP,)),
        ],
        compiler_params=pltpu.CompilerParams(collective_id=0),
    )(x, Wq, K_ext, V_ext, Wo)


# device time: 71717 ns/iter; 1.7368x vs baseline; 1.2394x over previous
import jax
import jax.numpy as jnp
from jax import lax
from jax.experimental import pallas as pl
from jax.experimental.pallas import tpu as pltpu

N_DEV = 8
HQ_PER = 8
DH = 128
SQ = 1024
D_MODEL = 1024
BLK = 64
N_CLASS = 4
BLOCKS_PER_CLASS = 4
CHUNK = SQ // N_DEV
HALF = D_MODEL // 2
SCALE = 0.08838834764831843
N_HOP = N_DEV - 1
BF16 = jnp.bfloat16
F32 = jnp.float32


def kernel(x, Wq, K_ext, V_ext, Wo):
    def body(x_ref, wq_ref, k_hbm, v_hbm, wo_ref, out_ref,
             k_ref, v_ref, q_ref, ctx_ref, wo_bf_ref,
             stage_r_ref, stage_l_ref, recv_r_ref, recv_l_ref,
             ag_r_ref, ag_l_ref, ag_own_r_ref, ag_own_l_ref,
             copy_sems, send_r_sems, recv_r_sems, send_l_sems, recv_l_sems):
        me = lax.axis_index("i")
        left = lax.rem(me + N_DEV - 1, N_DEV)
        right = lax.rem(me + 1, N_DEV)
        h0 = me * HQ_PER

        def mod(v):
            return lax.rem(v + 2 * N_DEV, N_DEV)

        copies = []
        for h in range(HQ_PER):
            kc = pltpu.make_async_copy(
                k_hbm.at[0, :, h0 + h, :], k_ref.at[h], copy_sems.at[h])
            vc = pltpu.make_async_copy(
                v_hbm.at[0, :, h0 + h, :], v_ref.at[h],
                copy_sems.at[HQ_PER + h])
            kc.start()
            vc.start()
            copies += [kc, vc]

        barrier = pltpu.get_barrier_semaphore()
        for nbr in (left, right):
            pl.semaphore_signal(barrier, inc=1, device_id=(nbr,),
                                device_id_type=pl.DeviceIdType.MESH)
        pl.semaphore_wait(barrier, 2)

        wo_bf_ref[:, :] = wo_ref[:, :].astype(BF16)

        q_ref[:, :] = jnp.dot(
            x_ref[0].astype(BF16), wq_ref[:, :].astype(BF16),
            preferred_element_type=F32).astype(BF16)

        for c in copies:
            c.wait()

        for c in range(N_CLASS):
            rows = [(c + N_CLASS * b) * BLK for b in range(BLOCKS_PER_CLASS)]
            for h in range(HQ_PER):
                qc = jnp.concatenate(
                    [q_ref[r:r + BLK, h * DH:(h + 1) * DH] for r in rows], 0)
                kc = jnp.concatenate(
                    [k_ref[h, r:r + BLK, :] for r in rows], 0).astype(BF16)
                vc = jnp.concatenate(
                    [v_ref[h, r:r + BLK, :] for r in rows], 0).astype(BF16)
                s = lax.dot_general(
                    qc, kc, (((1,), (1,)), ((), ())),
                    preferred_element_type=F32) * SCALE
                m = jnp.max(s, axis=1, keepdims=True)
                e = jnp.exp(s - m)
                w = (e / jnp.sum(e, axis=1, keepdims=True)).astype(BF16)
                ctx = jnp.dot(w, vc, preferred_element_type=F32)
                for b, r in enumerate(rows):
                    ctx_ref[r:r + BLK, h * DH:(h + 1) * DH] = \
                        ctx[b * BLK:(b + 1) * BLK, :].astype(BF16)

        o_ref = out_ref.at[0]

        def gemm(chunk, col0):
            r0 = chunk * CHUNK
            o_ref[pl.ds(r0, CHUNK), col0:col0 + HALF] = jnp.dot(
                ctx_ref[pl.ds(r0, CHUNK), :], wo_bf_ref[:, col0:col0 + HALF],
                preferred_element_type=F32)

        def ring_copy(src, dst, nbr, ssem, rsem):
            return pltpu.make_async_remote_copy(
                src_ref=src, dst_ref=dst, send_sem=ssem, recv_sem=rsem,
                device_id=(nbr,), device_id_type=pl.DeviceIdType.MESH)

        def o_chunk(chunk, col0):
            return o_ref.at[pl.ds(chunk * CHUNK, CHUNK), col0:col0 + HALF]

        gemm(me, 0)
        gemm(me, HALF)
        stage_r_ref[0] = o_chunk(me, 0)[:, :].astype(BF16)
        stage_l_ref[0] = o_chunk(me, HALF)[:, :].astype(BF16)
        hop_r = ring_copy(stage_r_ref.at[0], recv_r_ref.at[0], right,
                          send_r_sems.at[0], recv_r_sems.at[0])
        hop_l = ring_copy(stage_l_ref.at[0], recv_l_ref.at[0], left,
                          send_l_sems.at[0], recv_l_sems.at[0])
        hop_r.start()
        hop_l.start()
        for s in range(N_HOP):
            c_r = mod(me - s - 1)
            c_l = mod(me + s + 1)
            gemm(c_r, 0)
            gemm(c_l, HALF)
            hop_r.wait()
            val_r = o_chunk(c_r, 0)[:, :] + recv_r_ref[s].astype(F32)
            o_chunk(c_r, 0)[:, :] = val_r
            hop_l.wait()
            val_l = o_chunk(c_l, HALF)[:, :] + recv_l_ref[s].astype(F32)
            o_chunk(c_l, HALF)[:, :] = val_l
            if s < N_HOP - 1:
                stage_r_ref[s + 1] = val_r.astype(BF16)
                stage_l_ref[s + 1] = val_l.astype(BF16)
                hop_r = ring_copy(stage_r_ref.at[s + 1], recv_r_ref.at[s + 1],
                                  right, send_r_sems.at[s + 1],
                                  recv_r_sems.at[s + 1])
                hop_l = ring_copy(stage_l_ref.at[s + 1], recv_l_ref.at[s + 1],
                                  left, send_l_sems.at[s + 1],
                                  recv_l_sems.at[s + 1])
                hop_r.start()
                hop_l.start()
            else:
                ag_own_r_ref[:, :] = val_r.astype(BF16)
                ag_own_l_ref[:, :] = val_l.astype(BF16)

        hop_r = ring_copy(ag_own_r_ref, ag_r_ref.at[0], right,
                          send_r_sems.at[N_HOP], recv_r_sems.at[N_HOP])
        hop_l = ring_copy(ag_own_l_ref, ag_l_ref.at[0], left,
                          send_l_sems.at[N_HOP], recv_l_sems.at[N_HOP])
        hop_r.start()
        hop_l.start()
        for s in range(N_HOP):
            hop_r.wait()
            hop_l.wait()
            if s < N_HOP - 1:
                hop_r = ring_copy(ag_r_ref.at[s], ag_r_ref.at[s + 1], right,
                                  send_r_sems.at[N_HOP + s + 1],
                                  recv_r_sems.at[N_HOP + s + 1])
                hop_l = ring_copy(ag_l_ref.at[s], ag_l_ref.at[s + 1], left,
                                  send_l_sems.at[N_HOP + s + 1],
                                  recv_l_sems.at[N_HOP + s + 1])
                hop_r.start()
                hop_l.start()
            o_chunk(mod(me - s), 0)[:, :] = ag_r_ref[s].astype(F32)
            o_chunk(mod(me + s), HALF)[:, :] = ag_l_ref[s].astype(F32)

    return pl.pallas_call(
        body,
        out_shape=jax.ShapeDtypeStruct((1, SQ, D_MODEL), jnp.float32),
        in_specs=[
            pl.BlockSpec(memory_space=pltpu.VMEM),
            pl.BlockSpec(memory_space=pltpu.VMEM),
            pl.BlockSpec(memory_space=pl.ANY),
            pl.BlockSpec(memory_space=pl.ANY),
            pl.BlockSpec(memory_space=pltpu.VMEM),
        ],
        out_specs=pl.BlockSpec(memory_space=pltpu.VMEM),
        scratch_shapes=[
            pltpu.VMEM((HQ_PER, SQ, DH), F32),
            pltpu.VMEM((HQ_PER, SQ, DH), F32),
            pltpu.VMEM((SQ, HQ_PER * DH), BF16),
            pltpu.VMEM((SQ, HQ_PER * DH), BF16),
            pltpu.VMEM((D_MODEL, D_MODEL), BF16),
            pltpu.VMEM((N_HOP, CHUNK, HALF), BF16),
            pltpu.VMEM((N_HOP, CHUNK, HALF), BF16),
            pltpu.VMEM((N_HOP, CHUNK, HALF), BF16),
            pltpu.VMEM((N_HOP, CHUNK, HALF), BF16),
            pltpu.VMEM((N_HOP, CHUNK, HALF), BF16),
            pltpu.VMEM((N_HOP, CHUNK, HALF), BF16),
            pltpu.VMEM((CHUNK, HALF), BF16),
            pltpu.VMEM((CHUNK, HALF), BF16),
            pltpu.SemaphoreType.DMA((2 * HQ_PER,)),
            pltpu.SemaphoreType.DMA((2 * N_HOP,)),
            pltpu.SemaphoreType.DMA((2 * N_HOP,)),
            pltpu.SemaphoreType.DMA((2 * N_HOP,)),
            pltpu.SemaphoreType.DMA((2 * N_HOP,)),
        ],
        compiler_params=pltpu.CompilerParams(collective_id=0),
    )(x, Wq, K_ext, V_ext, Wo)


# device time: 52145 ns/iter; 2.3886x vs baseline; 1.3753x over previous
import jax
import jax.numpy as jnp
from jax import lax
from jax.experimental import pallas as pl
from jax.experimental.pallas import tpu as pltpu

N_DEV = 8
HQ_PER = 8
DH = 128
SQ = 1024
D_MODEL = 1024
BLK = 64
N_CLASS = 4
BLOCKS_PER_CLASS = 4
CHUNK = SQ // N_DEV
HALF = D_MODEL // 2
SCALE = 0.08838834764831843
N_HOP = N_DEV - 1
BF16 = jnp.bfloat16
F32 = jnp.float32


def kernel(x, Wq, K_ext, V_ext, Wo):
    def body(x_ref, wq_ref, k_hbm, v_hbm, wo_ref, out_ref,
             k_ref, v_ref, q_ref, ctx_ref, wo_bf_ref,
             obf_a_ref, obf_b_ref, rs_a_ref, rs_b_ref, ag_a_ref, ag_b_ref,
             copy_sems, send_a_sems, recv_a_sems, send_b_sems, recv_b_sems):
        me = lax.axis_index("i")
        h0 = me * HQ_PER

        t = lax.rem(me, 4)
        b_x = jnp.where((t == 1) | (t == 2), 1, 0)
        b_y = jnp.where(t >= 2, 1, 0)
        b_z = lax.div(me, 4)
        p_x = me + 1 - 2 * lax.rem(me, 2)
        p_y = (me - t) + (3 - t)
        p_z = me + 4 - 8 * b_z
        dims_a = [(p_x, b_x), (p_y, b_y), (p_z, b_z)]
        dims_b = [(p_z, b_z), (p_x, b_x), (p_y, b_y)]

        copies = []
        for h in range(HQ_PER):
            kc = pltpu.make_async_copy(
                k_hbm.at[0, :, h0 + h, :], k_ref.at[h], copy_sems.at[h])
            vc = pltpu.make_async_copy(
                v_hbm.at[0, :, h0 + h, :], v_ref.at[h],
                copy_sems.at[HQ_PER + h])
            kc.start()
            vc.start()
            copies += [kc, vc]

        barrier = pltpu.get_barrier_semaphore()
        for nbr in (p_x, p_y, p_z):
            pl.semaphore_signal(barrier, inc=1, device_id=(nbr,),
                                device_id_type=pl.DeviceIdType.MESH)
        pl.semaphore_wait(barrier, 3)

        wo_bf_ref[:, :] = wo_ref[:, :].astype(BF16)

        q_ref[:, :] = jnp.dot(
            x_ref[0].astype(BF16), wq_ref[:, :].astype(BF16),
            preferred_element_type=F32).astype(BF16)

        for c in copies:
            c.wait()

        for c in range(N_CLASS):
            rows = [(c + N_CLASS * b) * BLK for b in range(BLOCKS_PER_CLASS)]
            for h in range(HQ_PER):
                qc = jnp.concatenate(
                    [q_ref[r:r + BLK, h * DH:(h + 1) * DH] for r in rows], 0)
                kc = jnp.concatenate(
                    [k_ref[h, r:r + BLK, :] for r in rows], 0).astype(BF16)
                vc = jnp.concatenate(
                    [v_ref[h, r:r + BLK, :] for r in rows], 0).astype(BF16)
                s = lax.dot_general(
                    qc, kc, (((1,), (1,)), ((), ())),
                    preferred_element_type=F32) * SCALE
                m = jnp.max(s, axis=1, keepdims=True)
                e = jnp.exp(s - m)
                w = (e / jnp.sum(e, axis=1, keepdims=True)).astype(BF16)
                ctx = jnp.dot(w, vc, preferred_element_type=F32)
                for b, r in enumerate(rows):
                    ctx_ref[r:r + BLK, h * DH:(h + 1) * DH] = \
                        ctx[b * BLK:(b + 1) * BLK, :].astype(BF16)

        o_ref = out_ref.at[0]

        def gemm_rows(r0, col0, nrows):
            o_ref[pl.ds(r0, nrows), col0:col0 + HALF] = jnp.dot(
                ctx_ref[pl.ds(r0, nrows), :], wo_bf_ref[:, col0:col0 + HALF],
                preferred_element_type=F32)

        def xchg(src, dst, nbr, ssem, rsem):
            return pltpu.make_async_remote_copy(
                src_ref=src, dst_ref=dst, send_sem=ssem, recv_sem=rsem,
                device_id=(nbr,), device_id_type=pl.DeviceIdType.MESH)

        SIZES = [512, 256, 128]
        REG = [0, 512, 768]

        snd_a0 = (1 - b_x) * 512
        snd_b0 = (1 - b_z) * 512
        gemm_rows(snd_a0, 0, 512)
        obf_a_ref[pl.ds(snd_a0, 512), :] = \
            o_ref[pl.ds(snd_a0, 512), 0:HALF].astype(BF16)
        gemm_rows(snd_b0, HALF, 512)
        obf_b_ref[pl.ds(snd_b0, 512), :] = \
            o_ref[pl.ds(snd_b0, 512), HALF:D_MODEL].astype(BF16)

        off_a = 0
        off_b = 0
        for r in range(3):
            sz = SIZES[r]
            reg = REG[r]
            pa, ba = dims_a[r]
            pb, bb = dims_b[r]
            snd_a = off_a + (1 - ba) * sz
            snd_b = off_b + (1 - bb) * sz
            kp_a = off_a + ba * sz
            kp_b = off_b + bb * sz
            hop_a = xchg(obf_a_ref.at[pl.ds(snd_a, sz), :],
                         rs_a_ref.at[pl.ds(reg, sz), :], pa,
                         send_a_sems.at[r], recv_a_sems.at[r])
            hop_b = xchg(obf_b_ref.at[pl.ds(snd_b, sz), :],
                         rs_b_ref.at[pl.ds(reg, sz), :], pb,
                         send_b_sems.at[r], recv_b_sems.at[r])
            hop_a.start()
            hop_b.start()
            if r == 0:
                gemm_rows(kp_a, 0, 512)
                gemm_rows(kp_b, HALF, 512)
            hop_a.wait()
            val = (o_ref[pl.ds(kp_a, sz), 0:HALF]
                   + rs_a_ref[reg:reg + sz, :].astype(F32))
            o_ref[pl.ds(kp_a, sz), 0:HALF] = val
            obf_a_ref[pl.ds(kp_a, sz), :] = val.astype(BF16)
            hop_b.wait()
            val = (o_ref[pl.ds(kp_b, sz), HALF:D_MODEL]
                   + rs_b_ref[reg:reg + sz, :].astype(F32))
            o_ref[pl.ds(kp_b, sz), HALF:D_MODEL] = val
            obf_b_ref[pl.ds(kp_b, sz), :] = val.astype(BF16)
            off_a = kp_a
            off_b = kp_b

        for j, r in enumerate([2, 1, 0]):
            sz = SIZES[r]
            reg = REG[r]
            pa, ba = dims_a[r]
            pb, bb = dims_b[r]
            base_a = off_a - ba * sz
            base_b = off_b - bb * sz
            prt_a = base_a + (1 - ba) * sz
            prt_b = base_b + (1 - bb) * sz
            hop_a = xchg(obf_a_ref.at[pl.ds(off_a, sz), :],
                         ag_a_ref.at[pl.ds(reg, sz), :], pa,
                         send_a_sems.at[3 + j], recv_a_sems.at[3 + j])
            hop_b = xchg(obf_b_ref.at[pl.ds(off_b, sz), :],
                         ag_b_ref.at[pl.ds(reg, sz), :], pb,
                         send_b_sems.at[3 + j], recv_b_sems.at[3 + j])
            hop_a.start()
            hop_b.start()
            hop_a.wait()
            hop_b.wait()
            obf_a_ref[pl.ds(prt_a, sz), :] = ag_a_ref[reg:reg + sz, :]
            obf_b_ref[pl.ds(prt_b, sz), :] = ag_b_ref[reg:reg + sz, :]
            o_ref[pl.ds(prt_a, sz), 0:HALF] = \
                ag_a_ref[reg:reg + sz, :].astype(F32)
            o_ref[pl.ds(prt_b, sz), HALF:D_MODEL] = \
                ag_b_ref[reg:reg + sz, :].astype(F32)
            off_a = base_a
            off_b = base_b

    return pl.pallas_call(
        body,
        out_shape=jax.ShapeDtypeStruct((1, SQ, D_MODEL), jnp.float32),
        in_specs=[
            pl.BlockSpec(memory_space=pltpu.VMEM),
            pl.BlockSpec(memory_space=pltpu.VMEM),
            pl.BlockSpec(memory_space=pl.ANY),
            pl.BlockSpec(memory_space=pl.ANY),
            pl.BlockSpec(memory_space=pltpu.VMEM),
        ],
        out_specs=pl.BlockSpec(memory_space=pltpu.VMEM),
        scratch_shapes=[
            pltpu.VMEM((HQ_PER, SQ, DH), F32),
            pltpu.VMEM((HQ_PER, SQ, DH), F32),
            pltpu.VMEM((SQ, HQ_PER * DH), BF16),
            pltpu.VMEM((SQ, HQ_PER * DH), BF16),
            pltpu.VMEM((D_MODEL, D_MODEL), BF16),
            pltpu.VMEM((SQ, HALF), BF16),
            pltpu.VMEM((SQ, HALF), BF16),
            pltpu.VMEM((896, HALF), BF16),
            pltpu.VMEM((896, HALF), BF16),
            pltpu.VMEM((896, HALF), BF16),
            pltpu.VMEM((896, HALF), BF16),
            pltpu.SemaphoreType.DMA((2 * HQ_PER,)),
            pltpu.SemaphoreType.DMA((6,)),
            pltpu.SemaphoreType.DMA((6,)),
            pltpu.SemaphoreType.DMA((6,)),
            pltpu.SemaphoreType.DMA((6,)),
        ],
        compiler_params=pltpu.CompilerParams(collective_id=0),
    )(x, Wq, K_ext, V_ext, Wo)


# device time: 51831 ns/iter; 2.4031x vs baseline; 1.0061x over previous
import jax
import jax.numpy as jnp
from jax import lax
from jax.experimental import pallas as pl
from jax.experimental.pallas import tpu as pltpu

N_DEV = 8
HQ_PER = 8
DH = 128
SQ = 1024
D_MODEL = 1024
BLK = 64
N_CLASS = 4
BLOCKS_PER_CLASS = 4
CHUNK = SQ // N_DEV
HALF = D_MODEL // 2
SCALE = 0.08838834764831843
N_HOP = N_DEV - 1
BF16 = jnp.bfloat16
F32 = jnp.float32


def kernel(x, Wq, K_ext, V_ext, Wo):
    def body(x_ref, wq_ref, k_hbm, v_hbm, wo_ref, out_ref,
             k_ref, v_ref, q_ref, ctx_ref, wo_bf_ref,
             obf_a_ref, obf_b_ref, rs_a_ref, rs_b_ref, ag_a_ref, ag_b_ref,
             copy_sems, send_a_sems, recv_a_sems, send_b_sems, recv_b_sems):
        me = lax.axis_index("i")
        h0 = me * HQ_PER

        t = lax.rem(me, 4)
        b_x = jnp.where((t == 1) | (t == 2), 1, 0)
        b_y = jnp.where(t >= 2, 1, 0)
        b_z = lax.div(me, 4)
        p_x = me + 1 - 2 * lax.rem(me, 2)
        p_y = (me - t) + (3 - t)
        p_z = me + 4 - 8 * b_z
        dims_a = [(p_x, b_x), (p_y, b_y), (p_z, b_z)]
        dims_b = [(p_z, b_z), (p_x, b_x), (p_y, b_y)]

        copies = []
        for h in range(HQ_PER):
            kc = pltpu.make_async_copy(
                k_hbm.at[0, :, h0 + h, :], k_ref.at[h], copy_sems.at[h])
            vc = pltpu.make_async_copy(
                v_hbm.at[0, :, h0 + h, :], v_ref.at[h],
                copy_sems.at[HQ_PER + h])
            kc.start()
            vc.start()
            copies += [kc, vc]

        barrier = pltpu.get_barrier_semaphore()
        for nbr in (p_x, p_y, p_z):
            pl.semaphore_signal(barrier, inc=1, device_id=(nbr,),
                                device_id_type=pl.DeviceIdType.MESH)
        pl.semaphore_wait(barrier, 3)

        wo_bf_ref[:, :] = wo_ref[:, :].astype(BF16)

        q_ref[:, :] = jnp.dot(
            x_ref[0].astype(BF16), wq_ref[:, :].astype(BF16),
            preferred_element_type=F32).astype(BF16)

        for c in copies:
            c.wait()

        for c in range(N_CLASS):
            rows = [(c + N_CLASS * b) * BLK for b in range(BLOCKS_PER_CLASS)]
            for h in range(HQ_PER):
                qc = jnp.concatenate(
                    [q_ref[r:r + BLK, h * DH:(h + 1) * DH] for r in rows], 0)
                kc = jnp.concatenate(
                    [k_ref[h, r:r + BLK, :] for r in rows], 0).astype(BF16)
                vc = jnp.concatenate(
                    [v_ref[h, r:r + BLK, :] for r in rows], 0).astype(BF16)
                s = lax.dot_general(
                    qc, kc, (((1,), (1,)), ((), ())),
                    preferred_element_type=F32) * SCALE
                m = jnp.max(s, axis=1, keepdims=True)
                e = jnp.exp(s - m)
                w = (e / jnp.sum(e, axis=1, keepdims=True)).astype(BF16)
                ctx = jnp.dot(w, vc, preferred_element_type=F32)
                for b, r in enumerate(rows):
                    ctx_ref[r:r + BLK, h * DH:(h + 1) * DH] = \
                        ctx[b * BLK:(b + 1) * BLK, :].astype(BF16)

        o_ref = out_ref.at[0]

        def gemm_rows(r0, col0, nrows):
            o_ref[pl.ds(r0, nrows), col0:col0 + HALF] = jnp.dot(
                ctx_ref[pl.ds(r0, nrows), :], wo_bf_ref[:, col0:col0 + HALF],
                preferred_element_type=F32)

        def xchg(src, dst, nbr, ssem, rsem):
            return pltpu.make_async_remote_copy(
                src_ref=src, dst_ref=dst, send_sem=ssem, recv_sem=rsem,
                device_id=(nbr,), device_id_type=pl.DeviceIdType.MESH)

        SIZES = [512, 256, 128]
        REG = [0, 512, 768]

        snd_a0 = (1 - b_x) * 512
        snd_b0 = (1 - b_z) * 512
        gemm_rows(snd_a0, 0, 512)
        obf_a_ref[pl.ds(snd_a0, 512), :] = \
            o_ref[pl.ds(snd_a0, 512), 0:HALF].astype(BF16)
        gemm_rows(snd_b0, HALF, 512)
        obf_b_ref[pl.ds(snd_b0, 512), :] = \
            o_ref[pl.ds(snd_b0, 512), HALF:D_MODEL].astype(BF16)

        kp_a = []
        kp_b = []
        snd_a = []
        snd_b = []
        off_a = 0
        off_b = 0
        for r in range(3):
            sz = SIZES[r]
            ba = dims_a[r][1]
            bb = dims_b[r][1]
            snd_a.append(off_a + (1 - ba) * sz)
            snd_b.append(off_b + (1 - bb) * sz)
            kp_a.append(off_a + ba * sz)
            kp_b.append(off_b + bb * sz)
            off_a = kp_a[r]
            off_b = kp_b[r]
        ag_src_a = []
        ag_src_b = []
        prt_a = []
        prt_b = []
        for j, r in enumerate([2, 1, 0]):
            sz = SIZES[r]
            ba = dims_a[r][1]
            bb = dims_b[r][1]
            ag_src_a.append(off_a)
            ag_src_b.append(off_b)
            base_a = off_a - ba * sz
            base_b = off_b - bb * sz
            prt_a.append(base_a + (1 - ba) * sz)
            prt_b.append(base_b + (1 - bb) * sz)
            off_a = base_a
            off_b = base_b

        def rs_hop(sch, r):
            obf, rs, sems_s, sems_r, snd = (
                (obf_a_ref, rs_a_ref, send_a_sems, recv_a_sems, snd_a)
                if sch == 0 else
                (obf_b_ref, rs_b_ref, send_b_sems, recv_b_sems, snd_b))
            dims = dims_a if sch == 0 else dims_b
            return xchg(obf.at[pl.ds(snd[r], SIZES[r]), :],
                        rs.at[pl.ds(REG[r], SIZES[r]), :], dims[r][0],
                        sems_s.at[r], sems_r.at[r])

        def ag_hop(sch, j):
            r = 2 - j
            obf, ag, sems_s, sems_r, src = (
                (obf_a_ref, ag_a_ref, send_a_sems, recv_a_sems, ag_src_a)
                if sch == 0 else
                (obf_b_ref, ag_b_ref, send_b_sems, recv_b_sems, ag_src_b))
            dims = dims_a if sch == 0 else dims_b
            return xchg(obf.at[pl.ds(src[j], SIZES[r]), :],
                        ag.at[pl.ds(REG[r], SIZES[r]), :], dims[r][0],
                        sems_s.at[3 + j], sems_r.at[3 + j])

        hop_a = rs_hop(0, 0)
        hop_b = rs_hop(1, 0)
        hop_a.start()
        hop_b.start()
        gemm_rows(kp_a[0], 0, 512)
        gemm_rows(kp_b[0], HALF, 512)
        for r in range(3):
            sz = SIZES[r]
            reg = REG[r]
            hop_a.wait()
            val = (o_ref[pl.ds(kp_a[r], sz), 0:HALF]
                   + rs_a_ref[reg:reg + sz, :].astype(F32))
            o_ref[pl.ds(kp_a[r], sz), 0:HALF] = val
            obf_a_ref[pl.ds(kp_a[r], sz), :] = val.astype(BF16)
            hop_a = rs_hop(0, r + 1) if r < 2 else ag_hop(0, 0)
            hop_a.start()
            hop_b.wait()
            val = (o_ref[pl.ds(kp_b[r], sz), HALF:D_MODEL]
                   + rs_b_ref[reg:reg + sz, :].astype(F32))
            o_ref[pl.ds(kp_b[r], sz), HALF:D_MODEL] = val
            obf_b_ref[pl.ds(kp_b[r], sz), :] = val.astype(BF16)
            hop_b = rs_hop(1, r + 1) if r < 2 else ag_hop(1, 0)
            hop_b.start()

        for j, r in enumerate([2, 1, 0]):
            sz = SIZES[r]
            reg = REG[r]
            hop_a.wait()
            obf_a_ref[pl.ds(prt_a[j], sz), :] = ag_a_ref[reg:reg + sz, :]
            if j < 2:
                hop_a = ag_hop(0, j + 1)
                hop_a.start()
            o_ref[pl.ds(prt_a[j], sz), 0:HALF] = \
                ag_a_ref[reg:reg + sz, :].astype(F32)
            hop_b.wait()
            obf_b_ref[pl.ds(prt_b[j], sz), :] = ag_b_ref[reg:reg + sz, :]
            if j < 2:
                hop_b = ag_hop(1, j + 1)
                hop_b.start()
            o_ref[pl.ds(prt_b[j], sz), HALF:D_MODEL] = \
                ag_b_ref[reg:reg + sz, :].astype(F32)

    return pl.pallas_call(
        body,
        out_shape=jax.ShapeDtypeStruct((1, SQ, D_MODEL), jnp.float32),
        in_specs=[
            pl.BlockSpec(memory_space=pltpu.VMEM),
            pl.BlockSpec(memory_space=pltpu.VMEM),
            pl.BlockSpec(memory_space=pl.ANY),
            pl.BlockSpec(memory_space=pl.ANY),
            pl.BlockSpec(memory_space=pltpu.VMEM),
        ],
        out_specs=pl.BlockSpec(memory_space=pltpu.VMEM),
        scratch_shapes=[
            pltpu.VMEM((HQ_PER, SQ, DH), F32),
            pltpu.VMEM((HQ_PER, SQ, DH), F32),
            pltpu.VMEM((SQ, HQ_PER * DH), BF16),
            pltpu.VMEM((SQ, HQ_PER * DH), BF16),
            pltpu.VMEM((D_MODEL, D_MODEL), BF16),
            pltpu.VMEM((SQ, HALF), BF16),
            pltpu.VMEM((SQ, HALF), BF16),
            pltpu.VMEM((896, HALF), BF16),
            pltpu.VMEM((896, HALF), BF16),
            pltpu.VMEM((896, HALF), BF16),
            pltpu.VMEM((896, HALF), BF16),
            pltpu.SemaphoreType.DMA((2 * HQ_PER,)),
            pltpu.SemaphoreType.DMA((6,)),
            pltpu.SemaphoreType.DMA((6,)),
            pltpu.SemaphoreType.DMA((6,)),
            pltpu.SemaphoreType.DMA((6,)),
        ],
        compiler_params=pltpu.CompilerParams(collective_id=0),
    )(x, Wq, K_ext, V_ext, Wo)


# device time: 46156 ns/iter; 2.6986x vs baseline; 1.1230x over previous
import jax
import jax.numpy as jnp
from jax import lax
from jax.experimental import pallas as pl
from jax.experimental.pallas import tpu as pltpu

N_DEV = 8
HQ_PER = 8
DH = 128
SQ = 1024
D_MODEL = 1024
BLK = 64
N_CLASS = 4
BLOCKS_PER_CLASS = 4
SCALE = 0.08838834764831843
BF16 = jnp.bfloat16
F32 = jnp.float32

WID = [384, 384, 256]
COL0 = [0, 384, 768]
SIZES = [512, 256, 128]
REG = [0, 512, 768]


def kernel(x, Wq, K_ext, V_ext, Wo):
    def body(x_ref, wq_ref, k_hbm, v_hbm, wo_ref, out_ref,
             k_ref, v_ref, q_ref, ctx_ref, wo_bf_ref,
             obf0, obf1, obf2, rs0, rs1, rs2, ag0, ag1, ag2,
             copy_sems, send_sems, recv_sems):
        me = lax.axis_index("i")
        h0 = me * HQ_PER
        obf_refs = [obf0, obf1, obf2]
        rs_refs = [rs0, rs1, rs2]
        ag_refs = [ag0, ag1, ag2]

        t = lax.rem(me, 4)
        b_x = jnp.where((t == 1) | (t == 2), 1, 0)
        b_y = jnp.where(t >= 2, 1, 0)
        b_z = lax.div(me, 4)
        p_x = me + 1 - 2 * lax.rem(me, 2)
        p_y = (me - t) + (3 - t)
        p_z = me + 4 - 8 * b_z
        dx, dy, dz = (p_x, b_x), (p_y, b_y), (p_z, b_z)
        dims_s = [[dx, dy, dz], [dy, dz, dx], [dz, dx, dy]]

        copies = []
        for h in range(HQ_PER):
            kc = pltpu.make_async_copy(
                k_hbm.at[0, :, h0 + h, :], k_ref.at[h], copy_sems.at[h])
            vc = pltpu.make_async_copy(
                v_hbm.at[0, :, h0 + h, :], v_ref.at[h],
                copy_sems.at[HQ_PER + h])
            kc.start()
            vc.start()
            copies += [kc, vc]

        barrier = pltpu.get_barrier_semaphore()
        for nbr in (p_x, p_y, p_z):
            pl.semaphore_signal(barrier, inc=1, device_id=(nbr,),
                                device_id_type=pl.DeviceIdType.MESH)
        pl.semaphore_wait(barrier, 3)

        wo_bf_ref[:, :] = wo_ref[:, :].astype(BF16)

        q_ref[:, :] = jnp.dot(
            x_ref[0].astype(BF16), wq_ref[:, :].astype(BF16),
            preferred_element_type=F32).astype(BF16)

        for c in copies:
            c.wait()

        for c in range(N_CLASS):
            rows = [(c + N_CLASS * b) * BLK for b in range(BLOCKS_PER_CLASS)]
            for h in range(HQ_PER):
                qc = jnp.concatenate(
                    [q_ref[r:r + BLK, h * DH:(h + 1) * DH] for r in rows], 0)
                kc = jnp.concatenate(
                    [k_ref[h, r:r + BLK, :] for r in rows], 0).astype(BF16)
                vc = jnp.concatenate(
                    [v_ref[h, r:r + BLK, :] for r in rows], 0).astype(BF16)
                s = lax.dot_general(
                    qc, kc, (((1,), (1,)), ((), ())),
                    preferred_element_type=F32) * SCALE
                m = jnp.max(s, axis=1, keepdims=True)
                e = jnp.exp(s - m)
                w = (e / jnp.sum(e, axis=1, keepdims=True)).astype(BF16)
                ctx = jnp.dot(w, vc, preferred_element_type=F32)
                for b, r in enumerate(rows):
                    ctx_ref[r:r + BLK, h * DH:(h + 1) * DH] = \
                        ctx[b * BLK:(b + 1) * BLK, :].astype(BF16)

        o_ref = out_ref.at[0]

        def ocols(sch, r0, sz):
            return o_ref.at[pl.ds(r0, sz), COL0[sch]:COL0[sch] + WID[sch]]

        def gemm_rows(r0, sch, nrows):
            ocols(sch, r0, nrows)[:, :] = jnp.dot(
                ctx_ref[pl.ds(r0, nrows), :],
                wo_bf_ref[:, COL0[sch]:COL0[sch] + WID[sch]],
                preferred_element_type=F32)

        def xchg(src, dst, nbr, ssem, rsem):
            return pltpu.make_async_remote_copy(
                src_ref=src, dst_ref=dst, send_sem=ssem, recv_sem=rsem,
                device_id=(nbr,), device_id_type=pl.DeviceIdType.MESH)

        kp = [[], [], []]
        snd = [[], [], []]
        ag_src = [[], [], []]
        prt = [[], [], []]
        for s in range(3):
            off = 0
            for r in range(3):
                sz = SIZES[r]
                b = dims_s[s][r][1]
                snd[s].append(off + (1 - b) * sz)
                kp[s].append(off + b * sz)
                off = kp[s][r]
            for j, r in enumerate([2, 1, 0]):
                sz = SIZES[r]
                b = dims_s[s][r][1]
                ag_src[s].append(off)
                base = off - b * sz
                prt[s].append(base + (1 - b) * sz)
                off = base

        def rs_hop(s, r):
            return xchg(obf_refs[s].at[pl.ds(snd[s][r], SIZES[r]), :],
                        rs_refs[s].at[pl.ds(REG[r], SIZES[r]), :],
                        dims_s[s][r][0],
                        send_sems.at[s, r], recv_sems.at[s, r])

        def ag_hop(s, j):
            r = 2 - j
            return xchg(obf_refs[s].at[pl.ds(ag_src[s][j], SIZES[r]), :],
                        ag_refs[s].at[pl.ds(REG[r], SIZES[r]), :],
                        dims_s[s][r][0],
                        send_sems.at[s, 3 + j], recv_sems.at[s, 3 + j])

        hops = [None, None, None]
        for s in range(3):
            gemm_rows(snd[s][0], s, 512)
            obf_refs[s][pl.ds(snd[s][0], 512), :] = \
                ocols(s, snd[s][0], 512)[:, :].astype(BF16)
            hops[s] = rs_hop(s, 0)
            hops[s].start()
        for s in range(3):
            gemm_rows(kp[s][0], s, 512)
        for r in range(3):
            sz = SIZES[r]
            reg = REG[r]
            for s in range(3):
                hops[s].wait()
                val = (ocols(s, kp[s][r], sz)[:, :]
                       + rs_refs[s][reg:reg + sz, :].astype(F32))
                ocols(s, kp[s][r], sz)[:, :] = val
                obf_refs[s][pl.ds(kp[s][r], sz), :] = val.astype(BF16)
                hops[s] = rs_hop(s, r + 1) if r < 2 else ag_hop(s, 0)
                hops[s].start()

        for j, r in enumerate([2, 1, 0]):
            sz = SIZES[r]
            reg = REG[r]
            for s in range(3):
                hops[s].wait()
                obf_refs[s][pl.ds(prt[s][j], sz), :] = \
                    ag_refs[s][reg:reg + sz, :]
                if j < 2:
                    hops[s] = ag_hop(s, j + 1)
                    hops[s].start()
                ocols(s, prt[s][j], sz)[:, :] = \
                    ag_refs[s][reg:reg + sz, :].astype(F32)

    return pl.pallas_call(
        body,
        out_shape=jax.ShapeDtypeStruct((1, SQ, D_MODEL), jnp.float32),
        in_specs=[
            pl.BlockSpec(memory_space=pltpu.VMEM),
            pl.BlockSpec(memory_space=pltpu.VMEM),
            pl.BlockSpec(memory_space=pl.ANY),
            pl.BlockSpec(memory_space=pl.ANY),
            pl.BlockSpec(memory_space=pltpu.VMEM),
        ],
        out_specs=pl.BlockSpec(memory_space=pltpu.VMEM),
        scratch_shapes=[
            pltpu.VMEM((HQ_PER, SQ, DH), F32),
            pltpu.VMEM((HQ_PER, SQ, DH), F32),
            pltpu.VMEM((SQ, HQ_PER * DH), BF16),
            pltpu.VMEM((SQ, HQ_PER * DH), BF16),
            pltpu.VMEM((D_MODEL, D_MODEL), BF16),
            pltpu.VMEM((SQ, WID[0]), BF16),
            pltpu.VMEM((SQ, WID[1]), BF16),
            pltpu.VMEM((SQ, WID[2]), BF16),
            pltpu.VMEM((896, WID[0]), BF16),
            pltpu.VMEM((896, WID[1]), BF16),
            pltpu.VMEM((896, WID[2]), BF16),
            pltpu.VMEM((896, WID[0]), BF16),
            pltpu.VMEM((896, WID[1]), BF16),
            pltpu.VMEM((896, WID[2]), BF16),
            pltpu.SemaphoreType.DMA((2 * HQ_PER,)),
            pltpu.SemaphoreType.DMA((3, 6)),
            pltpu.SemaphoreType.DMA((3, 6)),
        ],
        compiler_params=pltpu.CompilerParams(collective_id=0),
    )(x, Wq, K_ext, V_ext, Wo)
